# Initial kernel scaffold; baseline (speedup 1.0000x reference)
import jax
import jax.numpy as jnp
from jax.experimental import pallas as pl


def kernel(points, freqs, cv):
    raise NotImplementedError("write your pallas kernel here")


# speedup vs baseline: 90.7379x; 90.7379x over previous
import functools

import jax
import jax.numpy as jnp
from jax import lax
from jax.experimental import pallas as pl
from jax.experimental.pallas import tpu as pltpu
from jax.experimental.pallas import tpu_sc as plsc

N = 32768
F = 6
C = 16
RES = 16
NVOL = 96
NB = 512
P = 1024
NW = 32
VPW = NVOL // NW
VOXELS = RES * RES * RES * C


def _encode_body(freqs_ref, pts_ref, i0_ref, i1_ref, w_ref, e_ref):
    pts = pts_ref[...]
    strides = (C, C * RES, C * RES * RES)
    for f in range(F):
        fp = pts * freqs_ref[f]
        s = jnp.sin(fp)
        c = jnp.cos(fp)
        for t, v in ((0, s), (1, c)):
            x = (v + 1.0) * (0.5 * (RES - 1))
            i0f = jnp.floor(x)
            w = x - i0f
            r = f * 6 + t * 3
            w_ref[r:r + 3, :] = w
            for a in range(3):
                i0a = i0f[a:a + 1, :].astype(jnp.int32) * strides[a]
                i1a = jnp.minimum(
                    i0f[a:a + 1, :] + 1.0, RES - 1.0
                ).astype(jnp.int32) * strides[a]
                i0_ref[r + a:r + a + 1, :] = i0a
                i1_ref[r + a:r + a + 1, :] = i1a
        e_ref[f * 2:f * 2 + 1, :] = s[0:1, :]
        e_ref[f * 2 + 1:f * 2 + 2, :] = c[0:1, :]


def _encode(pts_t, freqs):
    grid = (N // NB,)
    return pl.pallas_call(
        _encode_body,
        grid=grid,
        in_specs=[
            pl.BlockSpec(memory_space=pltpu.SMEM),
            pl.BlockSpec((3, NB), lambda i: (0, i)),
        ],
        out_specs=[
            pl.BlockSpec((36, NB), lambda i: (0, i)),
            pl.BlockSpec((36, NB), lambda i: (0, i)),
            pl.BlockSpec((36, NB), lambda i: (0, i)),
            pl.BlockSpec((12, NB), lambda i: (0, i)),
        ],
        out_shape=[
            jax.ShapeDtypeStruct((36, N), jnp.int32),
            jax.ShapeDtypeStruct((36, N), jnp.int32),
            jax.ShapeDtypeStruct((36, N), jnp.float32),
            jax.ShapeDtypeStruct((12, N), jnp.float32),
        ],
    )(freqs, pts_t)


@functools.partial(
    pl.kernel,
    mesh=plsc.VectorSubcoreMesh(core_axis_name="c", subcore_axis_name="s"),
    compiler_params=pltpu.CompilerParams(needs_layout_passes=False),
    out_type=jax.ShapeDtypeStruct((NVOL, C, N), jnp.float32),
    scratch_types=[
        pltpu.VMEM((VOXELS,), jnp.float32),
        pltpu.VMEM((P,), jnp.int32),
        pltpu.VMEM((P,), jnp.int32),
        pltpu.VMEM((P,), jnp.int32),
        pltpu.VMEM((P,), jnp.int32),
        pltpu.VMEM((P,), jnp.int32),
        pltpu.VMEM((P,), jnp.int32),
        pltpu.VMEM((P,), jnp.float32),
        pltpu.VMEM((P,), jnp.float32),
        pltpu.VMEM((P,), jnp.float32),
        pltpu.VMEM((P,), jnp.float32),
        pltpu.VMEM((C, P), jnp.float32),
        pltpu.SemaphoreType.DMA,
        pltpu.SemaphoreType.DMA,
    ],
)
def _sample(cv2, i0, i1, w, e, out, vol_v,
            ax0_v, ax1_v, ay0_v, ay1_v, az0_v, az1_v,
            wx_v, wy_v, wz_v, e_v, out_v, sem_in, sem_out):
    wid = lax.axis_index("s") * 2 + lax.axis_index("c")
    for vi in range(VPW):
        b = wid * VPW + vi
        f = b // 16
        g = (b // 8) % 2
        co = b % 8
        tx = co >> 2
        ty = (co >> 1) & 1
        tz = co & 1
        rx = f * 6 + tx * 3
        ry = f * 6 + ty * 3 + 1
        rz = f * 6 + tz * 3 + 2
        re = f * 2 + g
        pltpu.sync_copy(cv2.at[b], vol_v)

        def chunk_body(ci, _, rx=rx, ry=ry, rz=rz, re=re, b=b):
            n0 = ci * P
            sl = pl.ds(n0, P)
            cps = [
                pltpu.async_copy(i0.at[rx, sl], ax0_v, sem_in),
                pltpu.async_copy(i1.at[rx, sl], ax1_v, sem_in),
                pltpu.async_copy(i0.at[ry, sl], ay0_v, sem_in),
                pltpu.async_copy(i1.at[ry, sl], ay1_v, sem_in),
                pltpu.async_copy(i0.at[rz, sl], az0_v, sem_in),
                pltpu.async_copy(i1.at[rz, sl], az1_v, sem_in),
                pltpu.async_copy(w.at[rx, sl], wx_v, sem_in),
                pltpu.async_copy(w.at[ry, sl], wy_v, sem_in),
                pltpu.async_copy(w.at[rz, sl], wz_v, sem_in),
                pltpu.async_copy(e.at[re, sl], e_v, sem_in),
            ]
            for cp in cps:
                cp.wait()

            def grp_body(gi, _):
                p0 = gi * 16
                gsl = pl.ds(p0, 16)
                ax0 = ax0_v[gsl]
                ax1 = ax1_v[gsl]
                ay0 = ay0_v[gsl]
                ay1 = ay1_v[gsl]
                az0 = az0_v[gsl]
                az1 = az1_v[gsl]
                wx = wx_v[gsl]
                wy = wy_v[gsl]
                wz = wz_v[gsl]
                ev = e_v[gsl]
                zy00 = az0 + ay0
                zy01 = az0 + ay1
                zy10 = az1 + ay0
                zy11 = az1 + ay1
                k000 = zy00 + ax0
                k001 = zy00 + ax1
                k010 = zy01 + ax0
                k011 = zy01 + ax1
                k100 = zy10 + ax0
                k101 = zy10 + ax1
                k110 = zy11 + ax0
                k111 = zy11 + ax1
                for c in range(C):
                    c000 = plsc.load_gather(vol_v, [k000 + c])
                    c001 = plsc.load_gather(vol_v, [k001 + c])
                    c010 = plsc.load_gather(vol_v, [k010 + c])
                    c011 = plsc.load_gather(vol_v, [k011 + c])
                    c100 = plsc.load_gather(vol_v, [k100 + c])
                    c101 = plsc.load_gather(vol_v, [k101 + c])
                    c110 = plsc.load_gather(vol_v, [k110 + c])
                    c111 = plsc.load_gather(vol_v, [k111 + c])
                    c00 = c000 + wx * (c001 - c000)
                    c01 = c010 + wx * (c011 - c010)
                    c10 = c100 + wx * (c101 - c100)
                    c11 = c110 + wx * (c111 - c110)
                    c0 = c00 + wy * (c01 - c00)
                    c1 = c10 + wy * (c11 - c10)
                    res = c0 + wz * (c1 - c0) + ev
                    out_v[c, gsl] = res
                return 0

            lax.fori_loop(0, P // 16, grp_body, 0)
            pltpu.async_copy(
                out_v,
                out.at[b, :, sl],
                sem_out,
            ).wait()
            return 0

        lax.fori_loop(0, N // P, chunk_body, 0)


@jax.jit
def kernel(points, freqs, cv):
    pts_t = points.T
    cv2 = jnp.transpose(cv, (0, 2, 3, 4, 1)).reshape(NVOL, VOXELS)
    i0, i1, w, e = _encode(pts_t, freqs)
    out3 = _sample(cv2, i0, i1, w, e)
    return jnp.transpose(out3, (2, 0, 1)).reshape(N, NVOL * C)


# speedup vs baseline: 110.8230x; 1.2214x over previous
import functools

import jax
import jax.numpy as jnp
from jax import lax
from jax.experimental import pallas as pl
from jax.experimental.pallas import tpu as pltpu
from jax.experimental.pallas import tpu_sc as plsc

N = 32768
F = 6
C = 16
RES = 16
NVOL = 96
NB = 512
P = 1024
NW = 32
VPW = NVOL // NW
VOXELS = RES * RES * RES * C


def _encode_body(freqs_ref, pts_ref, i0_ref, i1_ref, w_ref, e_ref):
    pts = pts_ref[...]
    strides = (C, C * RES, C * RES * RES)
    for f in range(F):
        fp = pts * freqs_ref[f]
        s = jnp.sin(fp)
        c = jnp.cos(fp)
        for t, v in ((0, s), (1, c)):
            x = (v + 1.0) * (0.5 * (RES - 1))
            i0f = jnp.floor(x)
            w = x - i0f
            r = f * 6 + t * 3
            w_ref[r:r + 3, :] = w
            for a in range(3):
                i0a = i0f[a:a + 1, :].astype(jnp.int32) * strides[a]
                i1a = jnp.minimum(
                    i0f[a:a + 1, :] + 1.0, RES - 1.0
                ).astype(jnp.int32) * strides[a]
                i0_ref[r + a:r + a + 1, :] = i0a
                i1_ref[r + a:r + a + 1, :] = i1a
        e_ref[f * 2:f * 2 + 1, :] = s[0:1, :]
        e_ref[f * 2 + 1:f * 2 + 2, :] = c[0:1, :]


def _encode(pts_t, freqs):
    grid = (N // NB,)
    return pl.pallas_call(
        _encode_body,
        grid=grid,
        in_specs=[
            pl.BlockSpec(memory_space=pltpu.SMEM),
            pl.BlockSpec((3, NB), lambda i: (0, i)),
        ],
        out_specs=[
            pl.BlockSpec((36, NB), lambda i: (0, i)),
            pl.BlockSpec((36, NB), lambda i: (0, i)),
            pl.BlockSpec((36, NB), lambda i: (0, i)),
            pl.BlockSpec((12, NB), lambda i: (0, i)),
        ],
        out_shape=[
            jax.ShapeDtypeStruct((36, N), jnp.int32),
            jax.ShapeDtypeStruct((36, N), jnp.int32),
            jax.ShapeDtypeStruct((36, N), jnp.float32),
            jax.ShapeDtypeStruct((12, N), jnp.float32),
        ],
    )(freqs, pts_t)


@functools.partial(
    pl.kernel,
    mesh=plsc.VectorSubcoreMesh(core_axis_name="c", subcore_axis_name="s"),
    compiler_params=pltpu.CompilerParams(needs_layout_passes=False),
    out_type=jax.ShapeDtypeStruct((NVOL, C, N), jnp.float32),
    scratch_types=[
        pltpu.VMEM((VOXELS,), jnp.float32),
        pltpu.VMEM((P,), jnp.int32),
        pltpu.VMEM((P,), jnp.int32),
        pltpu.VMEM((P,), jnp.int32),
        pltpu.VMEM((P,), jnp.int32),
        pltpu.VMEM((P,), jnp.int32),
        pltpu.VMEM((P,), jnp.int32),
        pltpu.VMEM((P,), jnp.float32),
        pltpu.VMEM((P,), jnp.float32),
        pltpu.VMEM((P,), jnp.float32),
        pltpu.VMEM((P,), jnp.float32),
        pltpu.VMEM((C, P), jnp.float32),
        pltpu.SemaphoreType.DMA,
        pltpu.SemaphoreType.DMA,
    ],
)
def _sample(cv2, i0, i1, w, e, out, vol_v,
            ax0_v, ax1_v, ay0_v, ay1_v, az0_v, az1_v,
            wx_v, wy_v, wz_v, e_v, out_v, sem_in, sem_out):
    wid = lax.axis_index("s") * 2 + lax.axis_index("c")
    for vi in range(VPW):
        b = wid * VPW + vi
        f = b // 16
        g = (b // 8) % 2
        co = b % 8
        tx = co >> 2
        ty = (co >> 1) & 1
        tz = co & 1
        rx = f * 6 + tx * 3
        ry = f * 6 + ty * 3 + 1
        rz = f * 6 + tz * 3 + 2
        re = f * 2 + g
        pltpu.sync_copy(cv2.at[b], vol_v)

        def chunk_body(ci, _, rx=rx, ry=ry, rz=rz, re=re, b=b):
            n0 = ci * P
            sl = pl.ds(n0, P)
            cps = [
                pltpu.async_copy(i0.at[rx, sl], ax0_v, sem_in),
                pltpu.async_copy(i1.at[rx, sl], ax1_v, sem_in),
                pltpu.async_copy(i0.at[ry, sl], ay0_v, sem_in),
                pltpu.async_copy(i1.at[ry, sl], ay1_v, sem_in),
                pltpu.async_copy(i0.at[rz, sl], az0_v, sem_in),
                pltpu.async_copy(i1.at[rz, sl], az1_v, sem_in),
                pltpu.async_copy(w.at[rx, sl], wx_v, sem_in),
                pltpu.async_copy(w.at[ry, sl], wy_v, sem_in),
                pltpu.async_copy(w.at[rz, sl], wz_v, sem_in),
                pltpu.async_copy(e.at[re, sl], e_v, sem_in),
            ]
            for cp in cps:
                cp.wait()

            @plsc.parallel_loop(0, P // 16, 1, unroll=2)
            def grp_body(gi):
                p0 = gi * 16
                gsl = pl.ds(p0, 16)
                ax0 = ax0_v[gsl]
                ax1 = ax1_v[gsl]
                ay0 = ay0_v[gsl]
                ay1 = ay1_v[gsl]
                az0 = az0_v[gsl]
                az1 = az1_v[gsl]
                wx = wx_v[gsl]
                wy = wy_v[gsl]
                wz = wz_v[gsl]
                ev = e_v[gsl]
                zy00 = az0 + ay0
                zy01 = az0 + ay1
                zy10 = az1 + ay0
                zy11 = az1 + ay1
                k000 = zy00 + ax0
                k001 = zy00 + ax1
                k010 = zy01 + ax0
                k011 = zy01 + ax1
                k100 = zy10 + ax0
                k101 = zy10 + ax1
                k110 = zy11 + ax0
                k111 = zy11 + ax1
                for c in range(C):
                    c000 = plsc.load_gather(vol_v, [k000 + c])
                    c001 = plsc.load_gather(vol_v, [k001 + c])
                    c010 = plsc.load_gather(vol_v, [k010 + c])
                    c011 = plsc.load_gather(vol_v, [k011 + c])
                    c100 = plsc.load_gather(vol_v, [k100 + c])
                    c101 = plsc.load_gather(vol_v, [k101 + c])
                    c110 = plsc.load_gather(vol_v, [k110 + c])
                    c111 = plsc.load_gather(vol_v, [k111 + c])
                    c00 = c000 + wx * (c001 - c000)
                    c01 = c010 + wx * (c011 - c010)
                    c10 = c100 + wx * (c101 - c100)
                    c11 = c110 + wx * (c111 - c110)
                    c0 = c00 + wy * (c01 - c00)
                    c1 = c10 + wy * (c11 - c10)
                    res = c0 + wz * (c1 - c0) + ev
                    out_v[c, gsl] = res

            pltpu.async_copy(
                out_v,
                out.at[b, :, sl],
                sem_out,
            ).wait()
            return 0

        lax.fori_loop(0, N // P, chunk_body, 0)


@jax.jit
def kernel(points, freqs, cv):
    pts_t = points.T
    cv2 = jnp.transpose(cv, (0, 2, 3, 4, 1)).reshape(NVOL, VOXELS)
    i0, i1, w, e = _encode(pts_t, freqs)
    out3 = _sample(cv2, i0, i1, w, e)
    return jnp.transpose(out3, (2, 0, 1)).reshape(N, NVOL * C)


# speedup vs baseline: 249.6732x; 2.2529x over previous
import functools

import jax
import jax.numpy as jnp
from jax import lax
from jax.experimental import pallas as pl
from jax.experimental.pallas import tpu as pltpu
from jax.experimental.pallas import tpu_sc as plsc

N = 32768
F = 6
C = 16
RES = 16
NVOL = 96
NB = 512
P = 1024
NW = 32
VPW = NVOL // NW
ROWP = C + 1
VOXELS = RES * RES * RES * ROWP


def _encode_body(freqs_ref, pts_ref, i0_ref, i1_ref, w_ref, e_ref):
    pts = pts_ref[...]
    strides = (ROWP, ROWP * RES, ROWP * RES * RES)
    for f in range(F):
        fp = pts * freqs_ref[f]
        s = jnp.sin(fp)
        c = jnp.cos(fp)
        for t, v in ((0, s), (1, c)):
            x = (v + 1.0) * (0.5 * (RES - 1))
            i0f = jnp.floor(x)
            w = x - i0f
            r = f * 6 + t * 3
            w_ref[r:r + 3, :] = w
            for a in range(3):
                i0a = i0f[a:a + 1, :].astype(jnp.int32) * strides[a]
                i1a = jnp.minimum(
                    i0f[a:a + 1, :] + 1.0, RES - 1.0
                ).astype(jnp.int32) * strides[a]
                i0_ref[r + a:r + a + 1, :] = i0a
                i1_ref[r + a:r + a + 1, :] = i1a
        e_ref[f * 2:f * 2 + 1, :] = s[0:1, :]
        e_ref[f * 2 + 1:f * 2 + 2, :] = c[0:1, :]


def _encode(pts_t, freqs):
    grid = (N // NB,)
    return pl.pallas_call(
        _encode_body,
        grid=grid,
        in_specs=[
            pl.BlockSpec(memory_space=pltpu.SMEM),
            pl.BlockSpec((3, NB), lambda i: (0, i)),
        ],
        out_specs=[
            pl.BlockSpec((36, NB), lambda i: (0, i)),
            pl.BlockSpec((36, NB), lambda i: (0, i)),
            pl.BlockSpec((36, NB), lambda i: (0, i)),
            pl.BlockSpec((12, NB), lambda i: (0, i)),
        ],
        out_shape=[
            jax.ShapeDtypeStruct((36, N), jnp.int32),
            jax.ShapeDtypeStruct((36, N), jnp.int32),
            jax.ShapeDtypeStruct((36, N), jnp.float32),
            jax.ShapeDtypeStruct((12, N), jnp.float32),
        ],
    )(freqs, pts_t)


@functools.partial(
    pl.kernel,
    mesh=plsc.VectorSubcoreMesh(core_axis_name="c", subcore_axis_name="s"),
    compiler_params=pltpu.CompilerParams(needs_layout_passes=False),
    out_type=jax.ShapeDtypeStruct((NVOL, C, N), jnp.float32),
    scratch_types=[
        pltpu.VMEM((VOXELS,), jnp.float32),
        pltpu.VMEM((P,), jnp.int32),
        pltpu.VMEM((P,), jnp.int32),
        pltpu.VMEM((P,), jnp.int32),
        pltpu.VMEM((P,), jnp.int32),
        pltpu.VMEM((P,), jnp.int32),
        pltpu.VMEM((P,), jnp.int32),
        pltpu.VMEM((P,), jnp.float32),
        pltpu.VMEM((P,), jnp.float32),
        pltpu.VMEM((P,), jnp.float32),
        pltpu.VMEM((P,), jnp.float32),
        pltpu.VMEM((C, P), jnp.float32),
        pltpu.SemaphoreType.DMA,
        pltpu.SemaphoreType.DMA,
    ],
)
def _sample(cv2, i0, i1, w, e, out, vol_v,
            ax0_v, ax1_v, ay0_v, ay1_v, az0_v, az1_v,
            wx_v, wy_v, wz_v, e_v, out_v, sem_in, sem_out):
    wid = lax.axis_index("s") * 2 + lax.axis_index("c")
    for vi in range(VPW):
        b = wid * VPW + vi
        f = b // 16
        g = (b // 8) % 2
        co = b % 8
        tx = co >> 2
        ty = (co >> 1) & 1
        tz = co & 1
        rx = f * 6 + tx * 3
        ry = f * 6 + ty * 3 + 1
        rz = f * 6 + tz * 3 + 2
        re = f * 2 + g
        pltpu.sync_copy(cv2.at[b], vol_v)

        def chunk_body(ci, _, rx=rx, ry=ry, rz=rz, re=re, b=b):
            n0 = ci * P
            sl = pl.ds(n0, P)
            cps = [
                pltpu.async_copy(i0.at[rx, sl], ax0_v, sem_in),
                pltpu.async_copy(i1.at[rx, sl], ax1_v, sem_in),
                pltpu.async_copy(i0.at[ry, sl], ay0_v, sem_in),
                pltpu.async_copy(i1.at[ry, sl], ay1_v, sem_in),
                pltpu.async_copy(i0.at[rz, sl], az0_v, sem_in),
                pltpu.async_copy(i1.at[rz, sl], az1_v, sem_in),
                pltpu.async_copy(w.at[rx, sl], wx_v, sem_in),
                pltpu.async_copy(w.at[ry, sl], wy_v, sem_in),
                pltpu.async_copy(w.at[rz, sl], wz_v, sem_in),
                pltpu.async_copy(e.at[re, sl], e_v, sem_in),
            ]
            for cp in cps:
                cp.wait()

            @plsc.parallel_loop(0, P // 16, 1, unroll=2)
            def grp_body(gi):
                p0 = gi * 16
                gsl = pl.ds(p0, 16)
                ax0 = ax0_v[gsl]
                ax1 = ax1_v[gsl]
                ay0 = ay0_v[gsl]
                ay1 = ay1_v[gsl]
                az0 = az0_v[gsl]
                az1 = az1_v[gsl]
                wx = wx_v[gsl]
                wy = wy_v[gsl]
                wz = wz_v[gsl]
                ev = e_v[gsl]
                zy00 = az0 + ay0
                zy01 = az0 + ay1
                zy10 = az1 + ay0
                zy11 = az1 + ay1
                k000 = zy00 + ax0
                k001 = zy00 + ax1
                k010 = zy01 + ax0
                k011 = zy01 + ax1
                k100 = zy10 + ax0
                k101 = zy10 + ax1
                k110 = zy11 + ax0
                k111 = zy11 + ax1
                for c in range(C):
                    c000 = plsc.load_gather(vol_v, [k000 + c])
                    c001 = plsc.load_gather(vol_v, [k001 + c])
                    c010 = plsc.load_gather(vol_v, [k010 + c])
                    c011 = plsc.load_gather(vol_v, [k011 + c])
                    c100 = plsc.load_gather(vol_v, [k100 + c])
                    c101 = plsc.load_gather(vol_v, [k101 + c])
                    c110 = plsc.load_gather(vol_v, [k110 + c])
                    c111 = plsc.load_gather(vol_v, [k111 + c])
                    c00 = c000 + wx * (c001 - c000)
                    c01 = c010 + wx * (c011 - c010)
                    c10 = c100 + wx * (c101 - c100)
                    c11 = c110 + wx * (c111 - c110)
                    c0 = c00 + wy * (c01 - c00)
                    c1 = c10 + wy * (c11 - c10)
                    res = c0 + wz * (c1 - c0) + ev
                    out_v[c, gsl] = res

            pltpu.async_copy(
                out_v,
                out.at[b, :, sl],
                sem_out,
            ).wait()
            return 0

        lax.fori_loop(0, N // P, chunk_body, 0)


@jax.jit
def kernel(points, freqs, cv):
    pts_t = points.T
    cv2 = jnp.transpose(cv, (0, 2, 3, 4, 1)).reshape(NVOL, RES * RES * RES, C)
    cv2 = jnp.pad(cv2, ((0, 0), (0, 0), (0, ROWP - C))).reshape(NVOL, VOXELS)
    i0, i1, w, e = _encode(pts_t, freqs)
    out3 = _sample(cv2, i0, i1, w, e)
    return jnp.transpose(out3, (2, 0, 1)).reshape(N, NVOL * C)


# speedup vs baseline: 255.4593x; 1.0232x over previous
import functools

import jax
import jax.numpy as jnp
from jax import lax
from jax.experimental import pallas as pl
from jax.experimental.pallas import tpu as pltpu
from jax.experimental.pallas import tpu_sc as plsc

N = 32768
F = 6
C = 16
RES = 16
NVOL = 96
NB = 512
P = 1024
NW = 32
VPW = NVOL // NW
ROWP = C + 1
VOXELS = RES * RES * RES * ROWP


def _encode_body(freqs_ref, pts_ref, i0_ref, i1_ref, w_ref, e_ref):
    pts = pts_ref[...]
    strides = (ROWP, ROWP * RES, ROWP * RES * RES)
    for f in range(F):
        fp = pts * freqs_ref[f]
        s = jnp.sin(fp)
        c = jnp.cos(fp)
        for t, v in ((0, s), (1, c)):
            x = (v + 1.0) * (0.5 * (RES - 1))
            i0f = jnp.floor(x)
            w = x - i0f
            r = f * 6 + t * 3
            w_ref[r:r + 3, :] = w
            for a in range(3):
                i0a = i0f[a:a + 1, :].astype(jnp.int32) * strides[a]
                i1a = jnp.minimum(
                    i0f[a:a + 1, :] + 1.0, RES - 1.0
                ).astype(jnp.int32) * strides[a]
                i0_ref[r + a:r + a + 1, :] = i0a
                i1_ref[r + a:r + a + 1, :] = i1a
        e_ref[f * 2:f * 2 + 1, :] = s[0:1, :]
        e_ref[f * 2 + 1:f * 2 + 2, :] = c[0:1, :]


def _encode(pts_t, freqs):
    grid = (N // NB,)
    return pl.pallas_call(
        _encode_body,
        grid=grid,
        in_specs=[
            pl.BlockSpec(memory_space=pltpu.SMEM),
            pl.BlockSpec((3, NB), lambda i: (0, i)),
        ],
        out_specs=[
            pl.BlockSpec((36, NB), lambda i: (0, i)),
            pl.BlockSpec((36, NB), lambda i: (0, i)),
            pl.BlockSpec((36, NB), lambda i: (0, i)),
            pl.BlockSpec((12, NB), lambda i: (0, i)),
        ],
        out_shape=[
            jax.ShapeDtypeStruct((36, N), jnp.int32),
            jax.ShapeDtypeStruct((36, N), jnp.int32),
            jax.ShapeDtypeStruct((36, N), jnp.float32),
            jax.ShapeDtypeStruct((12, N), jnp.float32),
        ],
    )(freqs, pts_t)


@functools.partial(
    pl.kernel,
    mesh=plsc.VectorSubcoreMesh(core_axis_name="c", subcore_axis_name="s"),
    compiler_params=pltpu.CompilerParams(needs_layout_passes=False),
    out_type=jax.ShapeDtypeStruct((NVOL, C, N), jnp.float32),
    scratch_types=[
        pltpu.VMEM((VOXELS,), jnp.float32),
        pltpu.VMEM((P,), jnp.int32),
        pltpu.VMEM((P,), jnp.int32),
        pltpu.VMEM((P,), jnp.int32),
        pltpu.VMEM((P,), jnp.int32),
        pltpu.VMEM((P,), jnp.int32),
        pltpu.VMEM((P,), jnp.int32),
        pltpu.VMEM((P,), jnp.float32),
        pltpu.VMEM((P,), jnp.float32),
        pltpu.VMEM((P,), jnp.float32),
        pltpu.VMEM((P,), jnp.float32),
        pltpu.VMEM((C, P), jnp.float32),
        pltpu.SemaphoreType.DMA,
        pltpu.SemaphoreType.DMA,
    ],
)
def _sample(cv2, i0, i1, w, e, out, vol_v,
            ax0_v, ax1_v, ay0_v, ay1_v, az0_v, az1_v,
            wx_v, wy_v, wz_v, e_v, out_v, sem_in, sem_out):
    wid = lax.axis_index("s") * 2 + lax.axis_index("c")
    for vi in range(VPW):
        b = wid * VPW + vi
        f = b // 16
        g = (b // 8) % 2
        co = b % 8
        tx = co >> 2
        ty = (co >> 1) & 1
        tz = co & 1
        rx = f * 6 + tx * 3
        ry = f * 6 + ty * 3 + 1
        rz = f * 6 + tz * 3 + 2
        re = f * 2 + g
        pltpu.sync_copy(cv2.at[b], vol_v)

        def chunk_body(ci, _, rx=rx, ry=ry, rz=rz, re=re, b=b):
            n0 = ci * P
            sl = pl.ds(n0, P)
            cps = [
                pltpu.async_copy(i0.at[rx, sl], ax0_v, sem_in),
                pltpu.async_copy(i1.at[rx, sl], ax1_v, sem_in),
                pltpu.async_copy(i0.at[ry, sl], ay0_v, sem_in),
                pltpu.async_copy(i1.at[ry, sl], ay1_v, sem_in),
                pltpu.async_copy(i0.at[rz, sl], az0_v, sem_in),
                pltpu.async_copy(i1.at[rz, sl], az1_v, sem_in),
                pltpu.async_copy(w.at[rx, sl], wx_v, sem_in),
                pltpu.async_copy(w.at[ry, sl], wy_v, sem_in),
                pltpu.async_copy(w.at[rz, sl], wz_v, sem_in),
                pltpu.async_copy(e.at[re, sl], e_v, sem_in),
            ]
            for cp in cps:
                cp.wait()

            @plsc.parallel_loop(0, P // 16, 1, unroll=4)
            def grp_body(gi):
                p0 = gi * 16
                gsl = pl.ds(p0, 16)
                ax0 = ax0_v[gsl]
                ax1 = ax1_v[gsl]
                ay0 = ay0_v[gsl]
                ay1 = ay1_v[gsl]
                az0 = az0_v[gsl]
                az1 = az1_v[gsl]
                wx = wx_v[gsl]
                wy = wy_v[gsl]
                wz = wz_v[gsl]
                ev = e_v[gsl]
                zy00 = az0 + ay0
                zy01 = az0 + ay1
                zy10 = az1 + ay0
                zy11 = az1 + ay1
                k000 = zy00 + ax0
                k001 = zy00 + ax1
                k010 = zy01 + ax0
                k011 = zy01 + ax1
                k100 = zy10 + ax0
                k101 = zy10 + ax1
                k110 = zy11 + ax0
                k111 = zy11 + ax1
                for c in range(C):
                    c000 = plsc.load_gather(vol_v, [k000 + c])
                    c001 = plsc.load_gather(vol_v, [k001 + c])
                    c010 = plsc.load_gather(vol_v, [k010 + c])
                    c011 = plsc.load_gather(vol_v, [k011 + c])
                    c100 = plsc.load_gather(vol_v, [k100 + c])
                    c101 = plsc.load_gather(vol_v, [k101 + c])
                    c110 = plsc.load_gather(vol_v, [k110 + c])
                    c111 = plsc.load_gather(vol_v, [k111 + c])
                    c00 = c000 + wx * (c001 - c000)
                    c01 = c010 + wx * (c011 - c010)
                    c10 = c100 + wx * (c101 - c100)
                    c11 = c110 + wx * (c111 - c110)
                    c0 = c00 + wy * (c01 - c00)
                    c1 = c10 + wy * (c11 - c10)
                    res = c0 + wz * (c1 - c0) + ev
                    out_v[c, gsl] = res

            pltpu.async_copy(
                out_v,
                out.at[b, :, sl],
                sem_out,
            ).wait()
            return 0

        lax.fori_loop(0, N // P, chunk_body, 0)


@jax.jit
def kernel(points, freqs, cv):
    pts_t = points.T
    cv2 = jnp.transpose(cv, (0, 2, 3, 4, 1)).reshape(NVOL, RES * RES * RES, C)
    cv2 = jnp.pad(cv2, ((0, 0), (0, 0), (0, ROWP - C))).reshape(NVOL, VOXELS)
    i0, i1, w, e = _encode(pts_t, freqs)
    out3 = _sample(cv2, i0, i1, w, e)
    return jnp.transpose(out3, (2, 0, 1)).reshape(N, NVOL * C)


# speedup vs baseline: 313.7372x; 1.2281x over previous
import functools

import jax
import jax.numpy as jnp
from jax import lax
from jax.experimental import pallas as pl
from jax.experimental.pallas import tpu as pltpu
from jax.experimental.pallas import tpu_sc as plsc

N = 32768
F = 6
C = 16
RES = 16
NVOL = 96
NB = 512
P = 1024
NW = 32
VPW = NVOL // NW
ROWP = C
VOXELS = RES * RES * RES * ROWP


def _encode_body(freqs_ref, pts_ref, i0_ref, i1_ref, w_ref, e_ref):
    pts = pts_ref[...]
    strides = (ROWP, ROWP * RES, ROWP * RES * RES)
    for f in range(F):
        fp = pts * freqs_ref[f]
        s = jnp.sin(fp)
        c = jnp.cos(fp)
        for t, v in ((0, s), (1, c)):
            x = (v + 1.0) * (0.5 * (RES - 1))
            i0f = jnp.floor(x)
            w = x - i0f
            r = f * 6 + t * 3
            w_ref[r:r + 3, :] = w
            for a in range(3):
                i0a = i0f[a:a + 1, :].astype(jnp.int32) * strides[a]
                i1a = jnp.minimum(
                    i0f[a:a + 1, :] + 1.0, RES - 1.0
                ).astype(jnp.int32) * strides[a]
                i0_ref[r + a:r + a + 1, :] = i0a
                i1_ref[r + a:r + a + 1, :] = i1a
        e_ref[f * 2:f * 2 + 1, :] = s[0:1, :]
        e_ref[f * 2 + 1:f * 2 + 2, :] = c[0:1, :]


def _encode(pts_t, freqs):
    grid = (N // NB,)
    return pl.pallas_call(
        _encode_body,
        grid=grid,
        in_specs=[
            pl.BlockSpec(memory_space=pltpu.SMEM),
            pl.BlockSpec((3, NB), lambda i: (0, i)),
        ],
        out_specs=[
            pl.BlockSpec((36, NB), lambda i: (0, i)),
            pl.BlockSpec((36, NB), lambda i: (0, i)),
            pl.BlockSpec((36, NB), lambda i: (0, i)),
            pl.BlockSpec((12, NB), lambda i: (0, i)),
        ],
        out_shape=[
            jax.ShapeDtypeStruct((36, N), jnp.int32),
            jax.ShapeDtypeStruct((36, N), jnp.int32),
            jax.ShapeDtypeStruct((36, N), jnp.float32),
            jax.ShapeDtypeStruct((12, N), jnp.float32),
        ],
    )(freqs, pts_t)


@functools.partial(
    pl.kernel,
    mesh=plsc.VectorSubcoreMesh(core_axis_name="c", subcore_axis_name="s"),
    compiler_params=pltpu.CompilerParams(needs_layout_passes=False),
    out_type=jax.ShapeDtypeStruct((NVOL, C, N), jnp.float32),
    scratch_types=[
        pltpu.VMEM((VOXELS,), jnp.float32),
        pltpu.VMEM((P,), jnp.int32),
        pltpu.VMEM((P,), jnp.int32),
        pltpu.VMEM((P,), jnp.int32),
        pltpu.VMEM((P,), jnp.int32),
        pltpu.VMEM((P,), jnp.int32),
        pltpu.VMEM((P,), jnp.int32),
        pltpu.VMEM((P,), jnp.float32),
        pltpu.VMEM((P,), jnp.float32),
        pltpu.VMEM((P,), jnp.float32),
        pltpu.VMEM((P,), jnp.float32),
        pltpu.VMEM((C, P), jnp.float32),
        pltpu.SemaphoreType.DMA,
        pltpu.SemaphoreType.DMA,
    ],
)
def _sample(cv2, i0, i1, w, e, out, vol_v,
            ax0_v, ax1_v, ay0_v, ay1_v, az0_v, az1_v,
            wx_v, wy_v, wz_v, e_v, out_v, sem_in, sem_out):
    wid = lax.axis_index("s") * 2 + lax.axis_index("c")
    iot = lax.broadcasted_iota(jnp.int32, (16,), 0)
    for vi in range(VPW):
        b = wid * VPW + vi
        f = b // 16
        g = (b // 8) % 2
        co = b % 8
        tx = co >> 2
        ty = (co >> 1) & 1
        tz = co & 1
        rx = f * 6 + tx * 3
        ry = f * 6 + ty * 3 + 1
        rz = f * 6 + tz * 3 + 2
        re = f * 2 + g
        pltpu.sync_copy(cv2.at[b], vol_v)

        def chunk_body(ci, _, rx=rx, ry=ry, rz=rz, re=re, b=b):
            n0 = ci * P
            sl = pl.ds(n0, P)
            cps = [
                pltpu.async_copy(i0.at[rx, sl], ax0_v, sem_in),
                pltpu.async_copy(i1.at[rx, sl], ax1_v, sem_in),
                pltpu.async_copy(i0.at[ry, sl], ay0_v, sem_in),
                pltpu.async_copy(i1.at[ry, sl], ay1_v, sem_in),
                pltpu.async_copy(i0.at[rz, sl], az0_v, sem_in),
                pltpu.async_copy(i1.at[rz, sl], az1_v, sem_in),
                pltpu.async_copy(w.at[rx, sl], wx_v, sem_in),
                pltpu.async_copy(w.at[ry, sl], wy_v, sem_in),
                pltpu.async_copy(w.at[rz, sl], wz_v, sem_in),
                pltpu.async_copy(e.at[re, sl], e_v, sem_in),
            ]
            for cp in cps:
                cp.wait()

            @plsc.parallel_loop(0, P // 16, 1, unroll=4)
            def grp_body(gi):
                p0 = gi * 16
                gsl = pl.ds(p0, 16)
                ax0 = ax0_v[gsl]
                ax1 = ax1_v[gsl]
                ay0 = ay0_v[gsl]
                ay1 = ay1_v[gsl]
                az0 = az0_v[gsl]
                az1 = az1_v[gsl]
                wx = wx_v[gsl]
                wy = wy_v[gsl]
                wz = wz_v[gsl]
                ev = e_v[gsl]
                zy00 = az0 + ay0
                zy01 = az0 + ay1
                zy10 = az1 + ay0
                zy11 = az1 + ay1
                k000 = zy00 + ax0
                k001 = zy00 + ax1
                k010 = zy01 + ax0
                k011 = zy01 + ax1
                k100 = zy10 + ax0
                k101 = zy10 + ax1
                k110 = zy11 + ax0
                k111 = zy11 + ax1
                iop0 = iot + p0
                for j in range(C):
                    dg = (iot + j) & (C - 1)
                    c000 = plsc.load_gather(vol_v, [k000 + dg])
                    c001 = plsc.load_gather(vol_v, [k001 + dg])
                    c010 = plsc.load_gather(vol_v, [k010 + dg])
                    c011 = plsc.load_gather(vol_v, [k011 + dg])
                    c100 = plsc.load_gather(vol_v, [k100 + dg])
                    c101 = plsc.load_gather(vol_v, [k101 + dg])
                    c110 = plsc.load_gather(vol_v, [k110 + dg])
                    c111 = plsc.load_gather(vol_v, [k111 + dg])
                    c00 = c000 + wx * (c001 - c000)
                    c01 = c010 + wx * (c011 - c010)
                    c10 = c100 + wx * (c101 - c100)
                    c11 = c110 + wx * (c111 - c110)
                    c0 = c00 + wy * (c01 - c00)
                    c1 = c10 + wy * (c11 - c10)
                    res = c0 + wz * (c1 - c0) + ev
                    plsc.store_scatter(out_v, [dg, iop0], res)

            pltpu.async_copy(
                out_v,
                out.at[b, :, sl],
                sem_out,
            ).wait()
            return 0

        lax.fori_loop(0, N // P, chunk_body, 0)


@jax.jit
def kernel(points, freqs, cv):
    pts_t = points.T
    cv2 = jnp.transpose(cv, (0, 2, 3, 4, 1)).reshape(NVOL, VOXELS)
    i0, i1, w, e = _encode(pts_t, freqs)
    out3 = _sample(cv2, i0, i1, w, e)
    return jnp.transpose(out3, (2, 0, 1)).reshape(N, NVOL * C)


# speedup vs baseline: 318.9062x; 1.0165x over previous
import functools

import jax
import jax.numpy as jnp
from jax import lax
from jax.experimental import pallas as pl
from jax.experimental.pallas import tpu as pltpu
from jax.experimental.pallas import tpu_sc as plsc

N = 32768
F = 6
C = 16
RES = 16
NVOL = 96
NB = 512
P = 1024
NW = 32
VPW = NVOL // NW
ROWP = C
VOXELS = RES * RES * RES * ROWP


def _encode_body(freqs_ref, pts_ref, i0_ref, i1_ref, w_ref, e_ref):
    pts = pts_ref[...]
    strides = (ROWP, ROWP * RES, ROWP * RES * RES)
    for f in range(F):
        fp = pts * freqs_ref[f]
        s = jnp.sin(fp)
        c = jnp.cos(fp)
        for t, v in ((0, s), (1, c)):
            x = (v + 1.0) * (0.5 * (RES - 1))
            i0f = jnp.floor(x)
            w = x - i0f
            r = f * 6 + t * 3
            w_ref[r:r + 3, :] = w
            for a in range(3):
                i0a = i0f[a:a + 1, :].astype(jnp.int32) * strides[a]
                i1a = jnp.minimum(
                    i0f[a:a + 1, :] + 1.0, RES - 1.0
                ).astype(jnp.int32) * strides[a]
                i0_ref[r + a:r + a + 1, :] = i0a
                i1_ref[r + a:r + a + 1, :] = i1a
        e_ref[f * 2:f * 2 + 1, :] = s[0:1, :]
        e_ref[f * 2 + 1:f * 2 + 2, :] = c[0:1, :]


def _encode(pts_t, freqs):
    grid = (N // NB,)
    return pl.pallas_call(
        _encode_body,
        grid=grid,
        in_specs=[
            pl.BlockSpec(memory_space=pltpu.SMEM),
            pl.BlockSpec((3, NB), lambda i: (0, i)),
        ],
        out_specs=[
            pl.BlockSpec((36, NB), lambda i: (0, i)),
            pl.BlockSpec((36, NB), lambda i: (0, i)),
            pl.BlockSpec((36, NB), lambda i: (0, i)),
            pl.BlockSpec((12, NB), lambda i: (0, i)),
        ],
        out_shape=[
            jax.ShapeDtypeStruct((36, N), jnp.int32),
            jax.ShapeDtypeStruct((36, N), jnp.int32),
            jax.ShapeDtypeStruct((36, N), jnp.float32),
            jax.ShapeDtypeStruct((12, N), jnp.float32),
        ],
    )(freqs, pts_t)


@functools.partial(
    pl.kernel,
    mesh=plsc.VectorSubcoreMesh(core_axis_name="c", subcore_axis_name="s"),
    compiler_params=pltpu.CompilerParams(needs_layout_passes=False),
    out_type=jax.ShapeDtypeStruct((NVOL, C, N), jnp.float32),
    scratch_types=[
        pltpu.VMEM((VOXELS,), jnp.float32),
        pltpu.VMEM((P,), jnp.int32),
        pltpu.VMEM((P,), jnp.int32),
        pltpu.VMEM((P,), jnp.int32),
        pltpu.VMEM((P,), jnp.int32),
        pltpu.VMEM((P,), jnp.int32),
        pltpu.VMEM((P,), jnp.int32),
        pltpu.VMEM((P,), jnp.float32),
        pltpu.VMEM((P,), jnp.float32),
        pltpu.VMEM((P,), jnp.float32),
        pltpu.VMEM((P,), jnp.float32),
        pltpu.VMEM((C, P), jnp.float32),
        pltpu.VMEM((C, 16), jnp.int32),
        pltpu.SemaphoreType.DMA,
        pltpu.SemaphoreType.DMA,
    ],
)
def _sample(cv2, i0, i1, w, e, out, vol_v,
            ax0_v, ax1_v, ay0_v, ay1_v, az0_v, az1_v,
            wx_v, wy_v, wz_v, e_v, out_v, dg_v, sem_in, sem_out):
    wid = lax.axis_index("s") * 2 + lax.axis_index("c")
    iot = lax.broadcasted_iota(jnp.int32, (16,), 0)
    for j in range(C):
        dg_v[j, :] = (iot + j) & (C - 1)
    for vi in range(VPW):
        b = wid * VPW + vi
        f = b // 16
        g = (b // 8) % 2
        co = b % 8
        tx = co >> 2
        ty = (co >> 1) & 1
        tz = co & 1
        rx = f * 6 + tx * 3
        ry = f * 6 + ty * 3 + 1
        rz = f * 6 + tz * 3 + 2
        re = f * 2 + g
        pltpu.sync_copy(cv2.at[b], vol_v)

        def chunk_body(ci, _, rx=rx, ry=ry, rz=rz, re=re, b=b):
            n0 = ci * P
            sl = pl.ds(n0, P)
            cps = [
                pltpu.async_copy(i0.at[rx, sl], ax0_v, sem_in),
                pltpu.async_copy(i1.at[rx, sl], ax1_v, sem_in),
                pltpu.async_copy(i0.at[ry, sl], ay0_v, sem_in),
                pltpu.async_copy(i1.at[ry, sl], ay1_v, sem_in),
                pltpu.async_copy(i0.at[rz, sl], az0_v, sem_in),
                pltpu.async_copy(i1.at[rz, sl], az1_v, sem_in),
                pltpu.async_copy(w.at[rx, sl], wx_v, sem_in),
                pltpu.async_copy(w.at[ry, sl], wy_v, sem_in),
                pltpu.async_copy(w.at[rz, sl], wz_v, sem_in),
                pltpu.async_copy(e.at[re, sl], e_v, sem_in),
            ]
            for cp in cps:
                cp.wait()

            @plsc.parallel_loop(0, P // 16, 1, unroll=4)
            def grp_body(gi):
                p0 = gi * 16
                gsl = pl.ds(p0, 16)
                ax0 = ax0_v[gsl]
                ax1 = ax1_v[gsl]
                ay0 = ay0_v[gsl]
                ay1 = ay1_v[gsl]
                az0 = az0_v[gsl]
                az1 = az1_v[gsl]
                wx = wx_v[gsl]
                wy = wy_v[gsl]
                wz = wz_v[gsl]
                ev = e_v[gsl]
                gx0 = 1.0 - wx
                gy0 = 1.0 - wy
                gz0 = 1.0 - wz
                t00 = gz0 * gy0
                t01 = gz0 * wy
                t10 = wz * gy0
                t11 = wz * wy
                w000 = t00 * gx0
                w001 = t00 * wx
                w010 = t01 * gx0
                w011 = t01 * wx
                w100 = t10 * gx0
                w101 = t10 * wx
                w110 = t11 * gx0
                w111 = t11 * wx
                zy00 = az0 + ay0
                zy01 = az0 + ay1
                zy10 = az1 + ay0
                zy11 = az1 + ay1
                k000 = zy00 + ax0
                k001 = zy00 + ax1
                k010 = zy01 + ax0
                k011 = zy01 + ax1
                k100 = zy10 + ax0
                k101 = zy10 + ax1
                k110 = zy11 + ax0
                k111 = zy11 + ax1
                iop0 = iot + p0
                for j in range(C):
                    dg = dg_v[j, :]
                    c000 = plsc.load_gather(vol_v, [k000 + dg])
                    c001 = plsc.load_gather(vol_v, [k001 + dg])
                    c010 = plsc.load_gather(vol_v, [k010 + dg])
                    c011 = plsc.load_gather(vol_v, [k011 + dg])
                    c100 = plsc.load_gather(vol_v, [k100 + dg])
                    c101 = plsc.load_gather(vol_v, [k101 + dg])
                    c110 = plsc.load_gather(vol_v, [k110 + dg])
                    c111 = plsc.load_gather(vol_v, [k111 + dg])
                    acc0 = w000 * c000 + w001 * c001
                    acc1 = w010 * c010 + w011 * c011
                    acc2 = w100 * c100 + w101 * c101
                    acc3 = w110 * c110 + w111 * c111
                    res = ((acc0 + acc1) + (acc2 + acc3)) + ev
                    plsc.store_scatter(out_v, [dg, iop0], res)

            pltpu.async_copy(
                out_v,
                out.at[b, :, sl],
                sem_out,
            ).wait()
            return 0

        lax.fori_loop(0, N // P, chunk_body, 0)


@jax.jit
def kernel(points, freqs, cv):
    pts_t = points.T
    cv2 = jnp.transpose(cv, (0, 2, 3, 4, 1)).reshape(NVOL, VOXELS)
    i0, i1, w, e = _encode(pts_t, freqs)
    out3 = _sample(cv2, i0, i1, w, e)
    return jnp.transpose(out3, (2, 0, 1)).reshape(N, NVOL * C)
